# baseline (device time: 1484448 ns/iter reference)
import jax
import jax.numpy as jnp
from jax import lax
from jax.experimental import pallas as pl
from jax.experimental.pallas import tpu as pltpu

N_DEV = 32


def kernel(x, w_mat):
    m_tot, k_per = x.shape
    _, n = w_mat.shape
    ch = m_tot // N_DEV

    def body(x_ref, w_ref, out_ref, comm_ref, send_sems, recv_sems):
        my = lax.axis_index("i")
        left = (my - 1) % N_DEV
        right = (my + 1) % N_DEV

        barrier_sem = pltpu.get_barrier_semaphore()
        for nbr in [left, right]:
            pl.semaphore_signal(
                barrier_sem, inc=1,
                device_id=(nbr,), device_id_type=pl.DeviceIdType.MESH,
            )
        pl.semaphore_wait(barrier_sem, 2)

        c0 = (my - 1) % N_DEV
        comm_ref[0] = jnp.dot(
            x_ref[pl.ds(c0 * ch, ch), :], w_ref[:, :],
            preferred_element_type=jnp.float32,
        )

        for h in range(N_DEV - 1):
            send_slot = h % 2
            recv_slot = (h + 1) % 2
            rdma = pltpu.make_async_remote_copy(
                src_ref=comm_ref.at[send_slot],
                dst_ref=comm_ref.at[recv_slot],
                send_sem=send_sems.at[send_slot],
                recv_sem=recv_sems.at[recv_slot],
                device_id=(right,),
                device_id_type=pl.DeviceIdType.MESH,
            )
            rdma.start()
            rdma.wait()

            c = (my - h - 2) % N_DEV
            local = jnp.dot(
                x_ref[pl.ds(c * ch, ch), :], w_ref[:, :],
                preferred_element_type=jnp.float32,
            )
            if h < N_DEV - 2:
                comm_ref[recv_slot] = comm_ref[recv_slot] + local
            else:
                out_ref[:, :] = jnp.maximum(comm_ref[recv_slot] + local, 0.0)

    return pl.pallas_call(
        body,
        out_shape=jax.ShapeDtypeStruct((ch, n), jnp.float32),
        in_specs=[
            pl.BlockSpec(memory_space=pltpu.VMEM),
            pl.BlockSpec(memory_space=pltpu.VMEM),
        ],
        out_specs=pl.BlockSpec(memory_space=pltpu.VMEM),
        scratch_shapes=[
            pltpu.VMEM((2, ch, n), jnp.float32),
            pltpu.SemaphoreType.DMA((2,)),
            pltpu.SemaphoreType.DMA((2,)),
        ],
        compiler_params=pltpu.CompilerParams(collective_id=0),
    )(x, w_mat)


# device time: 1476154 ns/iter; 1.0056x vs baseline; 1.0056x over previous
import jax
import jax.numpy as jnp
from jax import lax
from jax.experimental import pallas as pl
from jax.experimental.pallas import tpu as pltpu

N_DEV = 32


def kernel(x, w_mat):
    m_tot, k_per = x.shape
    _, n = w_mat.shape
    ch = m_tot // N_DEV
    half = n // 2

    def body(x_ref, w_ref, out_ref,
             comm_r, comm_l, send_r, recv_r, send_l, recv_l,
             credit_r, credit_l):
        my = lax.axis_index("i")
        left = (my - 1) % N_DEV
        right = (my + 1) % N_DEV

        barrier_sem = pltpu.get_barrier_semaphore()
        for nbr in [left, right]:
            pl.semaphore_signal(
                barrier_sem, inc=1,
                device_id=(nbr,), device_id_type=pl.DeviceIdType.MESH,
            )
        pl.semaphore_wait(barrier_sem, 2)

        def partial_r(c):
            return jnp.dot(x_ref[pl.ds(c * ch, ch), :], w_ref[:, :half],
                           preferred_element_type=jnp.float32)

        def partial_l(c):
            return jnp.dot(x_ref[pl.ds(c * ch, ch), :], w_ref[:, half:],
                           preferred_element_type=jnp.float32)

        comm_r[0] = partial_r((my - 1) % N_DEV)
        comm_l[0] = partial_l((my + 1) % N_DEV)

        for h in range(N_DEV - 1):
            s = h % 2
            r = (h + 1) % 2
            if h >= 1:
                pl.semaphore_wait(credit_r, 1)
                pl.semaphore_wait(credit_l, 1)
            rdma_r = pltpu.make_async_remote_copy(
                src_ref=comm_r.at[s], dst_ref=comm_r.at[r],
                send_sem=send_r.at[s], recv_sem=recv_r.at[r],
                device_id=(right,), device_id_type=pl.DeviceIdType.MESH,
            )
            rdma_l = pltpu.make_async_remote_copy(
                src_ref=comm_l.at[s], dst_ref=comm_l.at[r],
                send_sem=send_l.at[s], recv_sem=recv_l.at[r],
                device_id=(left,), device_id_type=pl.DeviceIdType.MESH,
            )
            rdma_r.start()
            rdma_l.start()

            cr = (my - h - 2) % N_DEV
            cl = (my + h + 2) % N_DEV
            loc_r = partial_r(cr)
            loc_l = partial_l(cl)

            rdma_r.wait()
            rdma_l.wait()

            if h < N_DEV - 2:
                pl.semaphore_signal(
                    credit_r, inc=1,
                    device_id=(left,), device_id_type=pl.DeviceIdType.MESH,
                )
                pl.semaphore_signal(
                    credit_l, inc=1,
                    device_id=(right,), device_id_type=pl.DeviceIdType.MESH,
                )
                comm_r[r] = comm_r[r] + loc_r
                comm_l[r] = comm_l[r] + loc_l
            else:
                out_ref[:, :half] = jnp.maximum(comm_r[r] + loc_r, 0.0)
                out_ref[:, half:] = jnp.maximum(comm_l[r] + loc_l, 0.0)

    return pl.pallas_call(
        body,
        out_shape=jax.ShapeDtypeStruct((ch, n), jnp.float32),
        in_specs=[
            pl.BlockSpec(memory_space=pltpu.VMEM),
            pl.BlockSpec(memory_space=pltpu.VMEM),
        ],
        out_specs=pl.BlockSpec(memory_space=pltpu.VMEM),
        scratch_shapes=[
            pltpu.VMEM((2, ch, half), jnp.float32),
            pltpu.VMEM((2, ch, half), jnp.float32),
            pltpu.SemaphoreType.DMA((2,)),
            pltpu.SemaphoreType.DMA((2,)),
            pltpu.SemaphoreType.DMA((2,)),
            pltpu.SemaphoreType.DMA((2,)),
            pltpu.SemaphoreType.REGULAR,
            pltpu.SemaphoreType.REGULAR,
        ],
        compiler_params=pltpu.CompilerParams(collective_id=0),
    )(x, w_mat)


# device time: 785351 ns/iter; 1.8902x vs baseline; 1.8796x over previous
import jax
import jax.numpy as jnp
from jax import lax
from jax.experimental import pallas as pl
from jax.experimental.pallas import tpu as pltpu

N_DEV = 32

_PLANE = [(0, 0), (1, 0), (1, 1), (0, 1), (0, 2), (1, 2), (1, 3), (0, 3)]
_LOGICAL = {}
for _z in range(4):
    for _i, (_x, _y) in enumerate(_PLANE):
        _LOGICAL[(_x, _y, _z)] = _z * 8 + _i

_YZ_PATH = [
    (0, 0), (1, 0), (2, 0), (3, 0),
    (3, 1), (2, 1), (1, 1), (0, 1),
    (0, 2), (1, 2), (2, 2), (3, 2),
    (3, 3), (2, 3), (1, 3), (0, 3),
]
_CYCLE = [(0, y, z) for (y, z) in _YZ_PATH] + [
    (1, y, z) for (y, z) in reversed(_YZ_PATH)
]
RING = [_LOGICAL[c] for c in _CYCLE]
POS = [0] * N_DEV
for _p, _d in enumerate(RING):
    POS[_d] = _p


def kernel(x, w_mat):
    m_tot, k_per = x.shape
    _, n = w_mat.shape
    ch = m_tot // N_DEV
    half = n // 2

    def body(x_ref, w_ref, ring_ref, pos_ref, out_ref,
             comm_f, comm_b, send_f, recv_f, send_b, recv_b,
             credit_f, credit_b):
        my = lax.axis_index("i")

        ring_c = ring_ref[:, :]
        pos_c = pos_ref[:, :]
        idx_c = lax.broadcasted_iota(jnp.int32, (1, N_DEV), 1)

        def at_pos(p):
            return jnp.sum(jnp.where(idx_c == p, ring_c, 0))

        p = jnp.sum(jnp.where(idx_c == my, pos_c, 0))
        nxt = at_pos((p + 1) % N_DEV)
        prv = at_pos((p - 1) % N_DEV)

        barrier_sem = pltpu.get_barrier_semaphore()
        for nbr in [prv, nxt]:
            pl.semaphore_signal(
                barrier_sem, inc=1,
                device_id=(nbr,), device_id_type=pl.DeviceIdType.MESH,
            )
        pl.semaphore_wait(barrier_sem, 2)

        def partial_f(c):
            return jnp.dot(x_ref[pl.ds(c * ch, ch), :], w_ref[:, :half],
                           preferred_element_type=jnp.float32)

        def partial_b(c):
            return jnp.dot(x_ref[pl.ds(c * ch, ch), :], w_ref[:, half:],
                           preferred_element_type=jnp.float32)

        comm_f[0] = partial_f(at_pos((p - 1) % N_DEV))
        comm_b[0] = partial_b(at_pos((p + 1) % N_DEV))

        for h in range(N_DEV - 1):
            s = h % 2
            r = (h + 1) % 2
            if h >= 1:
                pl.semaphore_wait(credit_f, 1)
                pl.semaphore_wait(credit_b, 1)
            rdma_f = pltpu.make_async_remote_copy(
                src_ref=comm_f.at[s], dst_ref=comm_f.at[r],
                send_sem=send_f.at[s], recv_sem=recv_f.at[r],
                device_id=(nxt,), device_id_type=pl.DeviceIdType.MESH,
            )
            rdma_b = pltpu.make_async_remote_copy(
                src_ref=comm_b.at[s], dst_ref=comm_b.at[r],
                send_sem=send_b.at[s], recv_sem=recv_b.at[r],
                device_id=(prv,), device_id_type=pl.DeviceIdType.MESH,
            )
            rdma_f.start()
            rdma_b.start()

            cf = at_pos((p - h - 2) % N_DEV)
            cb = at_pos((p + h + 2) % N_DEV)
            loc_f = partial_f(cf)
            loc_b = partial_b(cb)

            rdma_f.wait()
            rdma_b.wait()

            if h < N_DEV - 2:
                pl.semaphore_signal(
                    credit_f, inc=1,
                    device_id=(prv,), device_id_type=pl.DeviceIdType.MESH,
                )
                pl.semaphore_signal(
                    credit_b, inc=1,
                    device_id=(nxt,), device_id_type=pl.DeviceIdType.MESH,
                )
                comm_f[r] = comm_f[r] + loc_f
                comm_b[r] = comm_b[r] + loc_b
            else:
                out_ref[:, :half] = jnp.maximum(comm_f[r] + loc_f, 0.0)
                out_ref[:, half:] = jnp.maximum(comm_b[r] + loc_b, 0.0)

    return pl.pallas_call(
        body,
        out_shape=jax.ShapeDtypeStruct((ch, n), jnp.float32),
        in_specs=[
            pl.BlockSpec(memory_space=pltpu.VMEM),
            pl.BlockSpec(memory_space=pltpu.VMEM),
            pl.BlockSpec(memory_space=pltpu.VMEM),
            pl.BlockSpec(memory_space=pltpu.VMEM),
        ],
        out_specs=pl.BlockSpec(memory_space=pltpu.VMEM),
        scratch_shapes=[
            pltpu.VMEM((2, ch, half), jnp.float32),
            pltpu.VMEM((2, ch, half), jnp.float32),
            pltpu.SemaphoreType.DMA((2,)),
            pltpu.SemaphoreType.DMA((2,)),
            pltpu.SemaphoreType.DMA((2,)),
            pltpu.SemaphoreType.DMA((2,)),
            pltpu.SemaphoreType.REGULAR,
            pltpu.SemaphoreType.REGULAR,
        ],
        compiler_params=pltpu.CompilerParams(collective_id=0),
    )(
        x,
        w_mat,
        jnp.array(RING, dtype=jnp.int32).reshape(1, N_DEV),
        jnp.array(POS, dtype=jnp.int32).reshape(1, N_DEV),
    )


# device time: 777483 ns/iter; 1.9093x vs baseline; 1.0101x over previous
import jax
import jax.numpy as jnp
from jax import lax
from jax.experimental import pallas as pl
from jax.experimental.pallas import tpu as pltpu

N_DEV = 32

_PLANE = [(0, 0), (1, 0), (1, 1), (0, 1), (0, 2), (1, 2), (1, 3), (0, 3)]
_LOGICAL = {}
for _z in range(4):
    for _i, (_x, _y) in enumerate(_PLANE):
        _LOGICAL[(_x, _y, _z)] = _z * 8 + _i

_YZ_PATH = [
    (0, 0), (1, 0), (2, 0), (3, 0),
    (3, 1), (2, 1), (1, 1), (0, 1),
    (0, 2), (1, 2), (2, 2), (3, 2),
    (3, 3), (2, 3), (1, 3), (0, 3),
]
_CYCLE = [(0, y, z) for (y, z) in _YZ_PATH] + [
    (1, y, z) for (y, z) in reversed(_YZ_PATH)
]
RING = [_LOGICAL[c] for c in _CYCLE]
POS = [0] * N_DEV
for _p, _d in enumerate(RING):
    POS[_d] = _p

N_STREAMS = 4


def kernel(x, w_mat):
    m_tot, k_per = x.shape
    _, n = w_mat.shape
    ch = m_tot // N_DEV
    q = n // N_STREAMS

    def body(x_ref, w_ref, ring_ref, pos_ref, out_ref, *scratch):
        comms = scratch[0:4]
        send_sems = scratch[4:8]
        recv_sems = scratch[8:12]
        credits = scratch[12:16]

        my = lax.axis_index("i")

        ring_c = ring_ref[:, :]
        pos_c = pos_ref[:, :]
        idx_c = lax.broadcasted_iota(jnp.int32, (1, N_DEV), 1)

        def at_pos(p):
            return jnp.sum(jnp.where(idx_c == p, ring_c, 0))

        p = jnp.sum(jnp.where(idx_c == my, pos_c, 0))
        nxt = at_pos((p + 1) % N_DEV)
        prv = at_pos((p - 1) % N_DEV)

        dest = [nxt, nxt, prv, prv]
        upstream = [prv, prv, nxt, nxt]

        barrier_sem = pltpu.get_barrier_semaphore()
        for nbr in [prv, nxt]:
            pl.semaphore_signal(
                barrier_sem, inc=1,
                device_id=(nbr,), device_id_type=pl.DeviceIdType.MESH,
            )
        pl.semaphore_wait(barrier_sem, 2)

        def partial(k, c):
            return jnp.dot(
                x_ref[pl.ds(c * ch, ch), :],
                w_ref[:, k * q:(k + 1) * q],
                preferred_element_type=jnp.float32,
            )

        def inbound_chunk(k, h):
            if k < 2:
                return at_pos((p - h - 2) % N_DEV)
            return at_pos((p + h + 2) % N_DEV)

        c_f = at_pos((p - 1) % N_DEV)
        c_b = at_pos((p + 1) % N_DEV)
        for k in range(N_STREAMS):
            comms[k][0] = partial(k, c_f if k < 2 else c_b)

        for h in range(N_DEV - 1):
            s = h % 2
            r = (h + 1) % 2
            rdmas = []
            for k in range(N_STREAMS):
                if h >= 1:
                    pl.semaphore_wait(credits[k], 1)
                rdma = pltpu.make_async_remote_copy(
                    src_ref=comms[k].at[s], dst_ref=comms[k].at[r],
                    send_sem=send_sems[k].at[s], recv_sem=recv_sems[k].at[r],
                    device_id=(dest[k],), device_id_type=pl.DeviceIdType.MESH,
                )
                rdma.start()
                rdmas.append(rdma)

            locs = [partial(k, inbound_chunk(k, h)) for k in range(N_STREAMS)]

            for k in range(N_STREAMS):
                rdmas[k].wait()
                if h < N_DEV - 2:
                    pl.semaphore_signal(
                        credits[k], inc=1,
                        device_id=(upstream[k],),
                        device_id_type=pl.DeviceIdType.MESH,
                    )
                    comms[k][r] = comms[k][r] + locs[k]
                else:
                    out_ref[:, k * q:(k + 1) * q] = jnp.maximum(
                        comms[k][r] + locs[k], 0.0
                    )

    return pl.pallas_call(
        body,
        out_shape=jax.ShapeDtypeStruct((ch, n), jnp.float32),
        in_specs=[
            pl.BlockSpec(memory_space=pltpu.VMEM),
            pl.BlockSpec(memory_space=pltpu.VMEM),
            pl.BlockSpec(memory_space=pltpu.VMEM),
            pl.BlockSpec(memory_space=pltpu.VMEM),
        ],
        out_specs=pl.BlockSpec(memory_space=pltpu.VMEM),
        scratch_shapes=(
            [pltpu.VMEM((2, ch, q), jnp.float32) for _ in range(4)]
            + [pltpu.SemaphoreType.DMA((2,)) for _ in range(4)]
            + [pltpu.SemaphoreType.DMA((2,)) for _ in range(4)]
            + [pltpu.SemaphoreType.REGULAR for _ in range(4)]
        ),
        compiler_params=pltpu.CompilerParams(collective_id=0),
    )(
        x,
        w_mat,
        jnp.array(RING, dtype=jnp.int32).reshape(1, N_DEV),
        jnp.array(POS, dtype=jnp.int32).reshape(1, N_DEV),
    )


# device time: 713885 ns/iter; 2.0794x vs baseline; 1.0891x over previous
import jax
import jax.numpy as jnp
from jax import lax
from jax.experimental import pallas as pl
from jax.experimental.pallas import tpu as pltpu

N_DEV = 32

_PLANE = [(0, 0), (1, 0), (1, 1), (0, 1), (0, 2), (1, 2), (1, 3), (0, 3)]
_LOGICAL = {}
for _z in range(4):
    for _i, (_x, _y) in enumerate(_PLANE):
        _LOGICAL[(_x, _y, _z)] = _z * 8 + _i

_YZ_PATH = [
    (0, 0), (1, 0), (2, 0), (3, 0),
    (3, 1), (2, 1), (1, 1), (0, 1),
    (0, 2), (1, 2), (2, 2), (3, 2),
    (3, 3), (2, 3), (1, 3), (0, 3),
]
_CYCLE = [(0, y, z) for (y, z) in _YZ_PATH] + [
    (1, y, z) for (y, z) in reversed(_YZ_PATH)
]
RING = [_LOGICAL[c] for c in _CYCLE]
POS = [0] * N_DEV
for _p, _d in enumerate(RING):
    POS[_d] = _p

N_STREAMS = 4


def kernel(x, w_mat):
    m_tot, k_per = x.shape
    _, n = w_mat.shape
    ch = m_tot // N_DEV
    q = n // N_STREAMS

    def body(x_ref, w_ref, ring_ref, pos_ref, out_ref, *scratch):
        comms = scratch[0:4]
        send_sems = scratch[4:8]
        recv_sems = scratch[8:12]
        credits = scratch[12:16]

        my = lax.axis_index("i")

        ring_c = ring_ref[:, :]
        pos_c = pos_ref[:, :]
        idx_c = lax.broadcasted_iota(jnp.int32, (1, N_DEV), 1)

        def at_pos(p):
            return jnp.sum(jnp.where(idx_c == p, ring_c, 0))

        p = jnp.sum(jnp.where(idx_c == my, pos_c, 0))
        nxt = at_pos((p + 1) % N_DEV)
        prv = at_pos((p - 1) % N_DEV)

        dest = [nxt, nxt, prv, prv]
        upstream = [prv, prv, nxt, nxt]

        barrier_sem = pltpu.get_barrier_semaphore()
        for nbr in [prv, nxt]:
            pl.semaphore_signal(
                barrier_sem, inc=1,
                device_id=(nbr,), device_id_type=pl.DeviceIdType.MESH,
            )
        pl.semaphore_wait(barrier_sem, 2)

        def partial(k, c):
            return jnp.dot(
                x_ref[pl.ds(c * ch, ch), :],
                w_ref[:, k * q:(k + 1) * q],
                preferred_element_type=jnp.float32,
            )

        def inbound_chunk(k, h):
            if k < 2:
                return at_pos((p - h - 2) % N_DEV)
            return at_pos((p + h + 2) % N_DEV)

        def make_rdma(k, h):
            s = h % 2
            r = (h + 1) % 2
            return pltpu.make_async_remote_copy(
                src_ref=comms[k].at[s], dst_ref=comms[k].at[r],
                send_sem=send_sems[k].at[s], recv_sem=recv_sems[k].at[r],
                device_id=(dest[k],), device_id_type=pl.DeviceIdType.MESH,
            )

        c_f = at_pos((p - 1) % N_DEV)
        c_b = at_pos((p + 1) % N_DEV)
        for k in range(N_STREAMS):
            comms[k][0] = partial(k, c_f if k < 2 else c_b)
        rdmas = [make_rdma(k, 0) for k in range(N_STREAMS)]
        for k in range(N_STREAMS):
            rdmas[k].start()
        loc_next = [partial(k, inbound_chunk(k, 0)) for k in range(N_STREAMS)]

        for h in range(N_DEV - 1):
            r = (h + 1) % 2
            last = h == N_DEV - 2
            for k in range(N_STREAMS):
                rdmas[k].wait()
                if not last:
                    pl.semaphore_signal(
                        credits[k], inc=1,
                        device_id=(upstream[k],),
                        device_id_type=pl.DeviceIdType.MESH,
                    )
                    comms[k][r] = comms[k][r] + loc_next[k]
                    pl.semaphore_wait(credits[k], 1)
                    rdmas[k] = make_rdma(k, h + 1)
                    rdmas[k].start()
                    loc_next[k] = partial(k, inbound_chunk(k, h + 1))
                else:
                    out_ref[:, k * q:(k + 1) * q] = jnp.maximum(
                        comms[k][r] + loc_next[k], 0.0
                    )

    return pl.pallas_call(
        body,
        out_shape=jax.ShapeDtypeStruct((ch, n), jnp.float32),
        in_specs=[
            pl.BlockSpec(memory_space=pltpu.VMEM),
            pl.BlockSpec(memory_space=pltpu.VMEM),
            pl.BlockSpec(memory_space=pltpu.VMEM),
            pl.BlockSpec(memory_space=pltpu.VMEM),
        ],
        out_specs=pl.BlockSpec(memory_space=pltpu.VMEM),
        scratch_shapes=(
            [pltpu.VMEM((2, ch, q), jnp.float32) for _ in range(4)]
            + [pltpu.SemaphoreType.DMA((2,)) for _ in range(4)]
            + [pltpu.SemaphoreType.DMA((2,)) for _ in range(4)]
            + [pltpu.SemaphoreType.REGULAR for _ in range(4)]
        ),
        compiler_params=pltpu.CompilerParams(collective_id=0),
    )(
        x,
        w_mat,
        jnp.array(RING, dtype=jnp.int32).reshape(1, N_DEV),
        jnp.array(POS, dtype=jnp.int32).reshape(1, N_DEV),
    )
